# Initial kernel scaffold; baseline (speedup 1.0000x reference)
import jax
import jax.numpy as jnp
from jax.experimental import pallas as pl


def kernel(inputs, emb_table, dense_W, dense_b):
    raise NotImplementedError("write your pallas kernel here")


# speedup vs baseline: 2.2134x; 2.2134x over previous
import functools

import jax
import jax.numpy as jnp
from jax import lax
from jax.experimental import pallas as pl
from jax.experimental.pallas import tpu as pltpu
from jax.experimental.pallas import tpu_sc as plsc

_VOCAB = 10
_ROWS_PAD = 16
_EMB_DIM = 8
_LANES = 16


_GATHER_DNUMS = lax.GatherDimensionNumbers(
    offset_dims=(), collapsed_slice_dims=(0,), start_index_map=(0,))


def _vgather(vec, idx):
    return lax.gather(vec, idx[:, None], _GATHER_DNUMS, slice_sizes=(1,),
                      mode=lax.GatherScatterMode.PROMISE_IN_BOUNDS)


def _bcast(vec, j):
    return _vgather(vec, jnp.full((_LANES,), j, jnp.int32))


@functools.lru_cache(maxsize=None)
def _build(batch):
    info = plsc.get_sparse_core_info()
    nc, ns = info.num_cores, info.num_subcores
    nw = nc * ns
    per_w = batch // nw
    assert batch % (8 * nw) == 0
    n_chunks = per_w // _LANES

    mesh = plsc.VectorSubcoreMesh(core_axis_name="c", subcore_axis_name="s")

    @functools.partial(
        pl.kernel,
        mesh=mesh,
        out_type=jax.ShapeDtypeStruct((batch,), jnp.float32),
        scratch_types=[
            pltpu.VMEM((per_w,), jnp.int32),
            pltpu.VMEM((_EMB_DIM * _LANES,), jnp.float32),
            pltpu.VMEM((_LANES,), jnp.float32),
            pltpu.VMEM((per_w,), jnp.float32),
        ],
    )
    def sc_kernel(x_hbm, embt_hbm, wb_hbm, out_hbm, x_v, emb_v, wb_v, out_v):
        wid = lax.axis_index("s") * nc + lax.axis_index("c")
        base = wid * per_w

        pltpu.sync_copy(x_hbm.at[pl.ds(base, per_w)], x_v)
        pltpu.sync_copy(embt_hbm, emb_v)
        pltpu.sync_copy(wb_hbm, wb_v)

        wb = wb_v[...]
        acc = _bcast(wb, _EMB_DIM)
        for j in range(_EMB_DIM):
            col = emb_v[pl.ds(j * _LANES, _LANES)]
            acc = acc + col * _bcast(wb, j)
        lut = 1.0 / (1.0 + jnp.exp(-acc))

        def body(i, lut):
            x = x_v[pl.ds(i * _LANES, _LANES)]
            idx = jnp.where((x >= 1) & (x <= _VOCAB), x, 0)
            out_v[pl.ds(i * _LANES, _LANES)] = _vgather(lut, idx)
            return lut

        lax.fori_loop(0, n_chunks, body, lut)
        pltpu.sync_copy(out_v, out_hbm.at[pl.ds(base, per_w)])

    return sc_kernel


def kernel(inputs, emb_table, dense_W, dense_b):
    batch = inputs.shape[0]
    x = inputs.reshape(batch)
    embt = jnp.pad(
        emb_table, ((0, _ROWS_PAD - emb_table.shape[0]), (0, 0))
    ).T.reshape(-1)
    wb = jnp.concatenate(
        [dense_W.reshape(-1), dense_b.reshape(-1),
         jnp.zeros((_LANES - _EMB_DIM - 1,), jnp.float32)]
    )
    out = _build(batch)(x, embt, wb)
    return out.reshape(batch, 1)


# speedup vs baseline: 2.3755x; 1.0732x over previous
import functools

import jax
import jax.numpy as jnp
from jax import lax
from jax.experimental import pallas as pl
from jax.experimental.pallas import tpu as pltpu
from jax.experimental.pallas import tpu_sc as plsc

_VOCAB = 10
_ROWS_PAD = 16
_EMB_DIM = 8
_LANES = 16
_TAB = _EMB_DIM * _ROWS_PAD

_GATHER_DNUMS = lax.GatherDimensionNumbers(
    offset_dims=(), collapsed_slice_dims=(0,), start_index_map=(0,))


def _vgather(vec, idx):
    return lax.gather(vec, idx[:, None], _GATHER_DNUMS, slice_sizes=(1,),
                      mode=lax.GatherScatterMode.PROMISE_IN_BOUNDS)


def _bcast(vec, j):
    return _vgather(vec, jnp.full((_LANES,), j, jnp.int32))


@functools.lru_cache(maxsize=None)
def _build(batch):
    info = plsc.get_sparse_core_info()
    nc, ns = info.num_cores, info.num_subcores
    nw = nc * ns
    per_w = batch // nw
    assert batch % (8 * nw) == 0
    n_chunks = per_w // _LANES

    mesh = plsc.VectorSubcoreMesh(core_axis_name="c", subcore_axis_name="s")

    @functools.partial(
        pl.kernel,
        mesh=mesh,
        out_type=jax.ShapeDtypeStruct((batch,), jnp.float32),
        scratch_types=[
            pltpu.VMEM((per_w,), jnp.int32),
            pltpu.VMEM((_TAB + _LANES,), jnp.float32),
            pltpu.VMEM((per_w,), jnp.float32),
            pltpu.SemaphoreType.DMA,
        ],
    )
    def sc_kernel(x_hbm, tab_hbm, out_hbm, x_v, tab_v, out_v, sem):
        wid = lax.axis_index("s") * nc + lax.axis_index("c")
        base = wid * per_w

        cp = pltpu.async_copy(x_hbm.at[pl.ds(base, per_w)], x_v, sem)
        pltpu.sync_copy(tab_hbm, tab_v)

        wb = tab_v[pl.ds(_TAB, _LANES)]
        acc = _bcast(wb, _EMB_DIM)
        for j in range(_EMB_DIM):
            col = tab_v[pl.ds(j * _LANES, _LANES)]
            acc = acc + col * _bcast(wb, j)
        lut = 1.0 / (1.0 + jnp.exp(-acc))

        cp.wait()
        for i in range(n_chunks):
            x = x_v[pl.ds(i * _LANES, _LANES)]
            idx = jnp.where((x >= 1) & (x <= _VOCAB), x, 0)
            out_v[pl.ds(i * _LANES, _LANES)] = _vgather(lut, idx)

        pltpu.sync_copy(out_v, out_hbm.at[pl.ds(base, per_w)])

    return sc_kernel


def kernel(inputs, emb_table, dense_W, dense_b):
    batch = inputs.shape[0]
    x = inputs.reshape(batch)
    embt = jnp.pad(
        emb_table, ((0, _ROWS_PAD - emb_table.shape[0]), (0, 0))
    ).T.reshape(-1)
    tab = jnp.concatenate(
        [embt, dense_W.reshape(-1), dense_b.reshape(-1),
         jnp.zeros((_LANES - _EMB_DIM - 1,), jnp.float32)]
    )
    out = _build(batch)(x, tab)
    return out.reshape(batch, 1)
